# baseline (speedup 1.0000x reference)
import functools

import jax
import jax.numpy as jnp
from jax import lax
from jax.experimental import pallas as pl
from jax.experimental.pallas import tpu as pltpu
from jax.experimental.pallas import tpu_sc as plsc

N = 10000
NP = 10240
E = 320000
D = 128
NC = 2
NS = 16
NW = NC * NS
K = 128
NB = -(-E // (NW * K))
E_PAD = NB * NW * K
SEG = NP // NS

_mesh = plsc.VectorSubcoreMesh(core_axis_name="c", subcore_axis_name="s")

def _neg2d(ref, nrows):
    ncol_chunks = ref.shape[1] // 16

    def body(i, _):
        r = i // ncol_chunks
        j = i % ncol_chunks
        ref[r, pl.ds(j * 16, 16)] = -ref[r, pl.ds(j * 16, 16)]
        return 0

    lax.fori_loop(0, nrows * ncol_chunks, body, 0)


def _iota_fill(idx_ref, base):
    def body(j, _):
        idx_ref[pl.ds(j * 16, 16)] = lax.iota(jnp.int32, 16) + base + j * 16
        return 0

    lax.fori_loop(0, idx_ref.shape[0] // 16, body, 0)


def _fill2d(ref, nrows, val):
    ncol_chunks = ref.shape[1] // 16

    def body(i, _):
        r = i // ncol_chunks
        j = i % ncol_chunks
        ref[r, pl.ds(j * 16, 16)] = jnp.full((16,), val, jnp.float32)
        return 0

    lax.fori_loop(0, nrows * ncol_chunks, body, 0)


SCRUB_R = 12288


@functools.partial(
    pl.kernel,
    mesh=_mesh,
    out_type=jax.ShapeDtypeStruct((8,), jnp.float32),
    scratch_types=[
        pltpu.VMEM((K,), jnp.int32),
        pltpu.VMEM((K, D), jnp.float32),
        pltpu.VMEM_SHARED((SCRUB_R, D), jnp.float32),
    ],
)
def _scrub_kernel(out_hbm, iot_v, z_v, sp_sh):
    c = lax.axis_index("c")
    s = lax.axis_index("s")
    _fill2d(z_v, K, 0.0)
    base = s * (SCRUB_R // NS)
    for t in range(SCRUB_R // NS // K):
        _iota_fill(iot_v, base + t * K)
        pltpu.sync_copy(z_v, sp_sh.at[iot_v])

    @pl.when(jnp.logical_and(c == 0, s == 0))
    def _():
        pltpu.sync_copy(z_v.at[0, pl.ds(0, 8)], out_hbm)


@functools.partial(
    pl.kernel,
    mesh=_mesh,
    out_type=jax.ShapeDtypeStruct((NC * NP, D), jnp.float32),
    scratch_types=[
        pltpu.VMEM((K,), jnp.int32),
        pltpu.VMEM((K,), jnp.int32),
        pltpu.VMEM((K, D), jnp.float32),
        pltpu.VMEM((K, D), jnp.float32),
        pltpu.VMEM_SHARED((NP, D), jnp.float32),
        pltpu.SemaphoreType.DMA,
    ],
)
def _deg_kernel(dst_hbm, out_hbm, idx_v, iot_v, ones_v, stg_v, acc_sh, sem):
    c = lax.axis_index("c")
    s = lax.axis_index("s")
    w = s * NC + c

    _fill2d(ones_v, K, 1.0)
    for t in range(SEG // K):
        _iota_fill(iot_v, s * SEG + t * K)
        pltpu.async_copy(acc_sh.at[iot_v], stg_v, sem).wait()
        _neg2d(stg_v, K)
        pltpu.sync_copy(stg_v, acc_sh.at[iot_v], add=True)
    plsc.subcore_barrier()

    def body(i, _):
        g = i * NW + w
        pltpu.sync_copy(dst_hbm.at[pl.ds(g * K, K)], idx_v)
        pltpu.sync_copy(ones_v, acc_sh.at[idx_v], add=True)
        return 0

    lax.fori_loop(0, NB, body, 0)
    plsc.subcore_barrier()

    for t in range(SEG // K):
        _iota_fill(iot_v, s * SEG + t * K)
        pltpu.async_copy(acc_sh.at[iot_v], stg_v, sem).wait()
        pltpu.sync_copy(stg_v, out_hbm.at[pl.ds(c * NP + s * SEG + t * K, K)])


@functools.partial(
    pl.kernel,
    mesh=_mesh,
    out_type=jax.ShapeDtypeStruct((NC * NP, D), jnp.float32),
    scratch_types=[
        pltpu.VMEM((K,), jnp.int32),
        pltpu.VMEM((K,), jnp.int32),
        pltpu.VMEM((K,), jnp.int32),
        pltpu.VMEM((K, D), jnp.float32),
        pltpu.VMEM_SHARED((NP, D), jnp.float32),
        pltpu.SemaphoreType.DMA,
    ],
)
def _spmm_kernel(y_hbm, src_hbm, dst_hbm, out_hbm, didx_v, sidx_v, iot_v, rows_v, acc_sh, sem):
    c = lax.axis_index("c")
    s = lax.axis_index("s")
    w = s * NC + c

    for t in range(SEG // K):
        _iota_fill(iot_v, s * SEG + t * K)
        pltpu.async_copy(acc_sh.at[iot_v], rows_v, sem).wait()
        _neg2d(rows_v, K)
        pltpu.sync_copy(rows_v, acc_sh.at[iot_v], add=True)
    plsc.subcore_barrier()

    def body(i, _):
        g = i * NW + w
        pltpu.sync_copy(dst_hbm.at[pl.ds(g * K, K)], didx_v)
        pltpu.sync_copy(src_hbm.at[pl.ds(g * K, K)], sidx_v)
        pltpu.async_copy(y_hbm.at[didx_v], rows_v, sem).wait()
        pltpu.sync_copy(rows_v, acc_sh.at[sidx_v], add=True)
        return 0

    lax.fori_loop(0, NB, body, 0)
    plsc.subcore_barrier()

    for t in range(SEG // K):
        _iota_fill(iot_v, s * SEG + t * K)
        pltpu.async_copy(acc_sh.at[iot_v], rows_v, sem).wait()
        pltpu.sync_copy(rows_v, out_hbm.at[pl.ds(c * NP + s * SEG + t * K, K)])


BM = 256
GRID = NP // BM


def _dinv_of(deg_ref):
    deg = deg_ref[0, :, 0:1] + deg_ref[1, :, 0:1] + 1.0
    return lax.rsqrt(deg)


def _tc1_body(deg_ref, x_ref, w1_ref, y_ref):
    dinv = _dinv_of(deg_ref)
    xw = jnp.dot(x_ref[...], w1_ref[...], preferred_element_type=jnp.float32,
                 precision=lax.Precision.HIGHEST)
    y_ref[...] = xw * dinv


def _tc2_body(deg_ref, y_ref, p_ref, b1_ref, g1_ref, bt1_ref, w2_ref, out_ref):
    dinv = _dinv_of(deg_ref)
    h = (y_ref[...] + p_ref[0] + p_ref[1]) * dinv + b1_ref[...]
    mu = jnp.mean(h, axis=1, keepdims=True)
    d = h - mu
    var = jnp.mean(d * d, axis=1, keepdims=True)
    hn = d * lax.rsqrt(var + 1e-5) * g1_ref[...] + bt1_ref[...]
    hr = jnp.maximum(hn, 0.0)
    out_ref[...] = jnp.dot(hr, w2_ref[...], preferred_element_type=jnp.float32,
                           precision=lax.Precision.HIGHEST) * dinv


def _tc3_body(deg_ref, y_ref, p_ref, b2_ref, out_ref):
    dinv = _dinv_of(deg_ref)
    out_ref[...] = (y_ref[...] + p_ref[0] + p_ref[1]) * dinv + b2_ref[...]


_deg_spec = pl.BlockSpec((NC, BM, D), lambda i: (0, i, 0))
_row_spec = pl.BlockSpec((BM, D), lambda i: (i, 0))
_p_spec = pl.BlockSpec((NC, BM, D), lambda i: (0, i, 0))
_w_spec = pl.BlockSpec((D, D), lambda i: (0, 0))
_v_spec = pl.BlockSpec((1, D), lambda i: (0, 0))
_out_sds = jax.ShapeDtypeStruct((NP, D), jnp.float32)

_DEBUG_JAX_SPMM = False


def kernel(x, edge_index, W1, b1, g1, beta1, W2, b2):
    src = jnp.pad(edge_index[0].astype(jnp.int32), (0, E_PAD - E), constant_values=N)
    dst = jnp.pad(edge_index[1].astype(jnp.int32), (0, E_PAD - E), constant_values=N)
    x_pad = jnp.pad(x, ((0, NP - N), (0, 0)))

    scrub = _scrub_kernel()
    dst = dst + (scrub[0] * 0.0).astype(jnp.int32)

    degp = _deg_kernel(dst).reshape(NC, NP, D)

    if _DEBUG_JAX_SPMM:
        def spmm(y):
            p = jnp.zeros((NP, D), jnp.float32).at[src].add(y[dst])
            return jnp.stack([p, jnp.zeros_like(p)])
    else:
        def spmm(y):
            return _spmm_kernel(y, src, dst).reshape(NC, NP, D)

    y1 = pl.pallas_call(
        _tc1_body,
        grid=(GRID,),
        in_specs=[_deg_spec, _row_spec, _w_spec],
        out_specs=_row_spec,
        out_shape=_out_sds,
    )(degp, x_pad, W1)

    p1 = spmm(y1)

    y2 = pl.pallas_call(
        _tc2_body,
        grid=(GRID,),
        in_specs=[_deg_spec, _row_spec, _p_spec, _v_spec, _v_spec, _v_spec, _w_spec],
        out_specs=_row_spec,
        out_shape=_out_sds,
    )(degp, y1, p1, b1.reshape(1, D), g1.reshape(1, D), beta1.reshape(1, D), W2)

    p2 = spmm(y2)

    out = pl.pallas_call(
        _tc3_body,
        grid=(GRID,),
        in_specs=[_deg_spec, _row_spec, _p_spec, _v_spec],
        out_specs=_row_spec,
        out_shape=_out_sds,
    )(degp, y2, p2, b2.reshape(1, D))

    return out[:N]


# speedup vs baseline: 1.2011x; 1.2011x over previous
import functools

import jax
import jax.numpy as jnp
from jax import lax
from jax.experimental import pallas as pl
from jax.experimental.pallas import tpu as pltpu
from jax.experimental.pallas import tpu_sc as plsc

N = 10000
NP = 10240
E = 320000
D = 128
NC = 2
NS = 16
NW = NC * NS
K = 128
NB = -(-E // (NW * K))
E_PAD = NB * NW * K
SEG = NP // NS

_mesh = plsc.VectorSubcoreMesh(core_axis_name="c", subcore_axis_name="s")

def _neg2d(ref, nrows):
    ncol_chunks = ref.shape[1] // 16

    def body(i, _):
        r = i // ncol_chunks
        j = i % ncol_chunks
        ref[r, pl.ds(j * 16, 16)] = -ref[r, pl.ds(j * 16, 16)]
        return 0

    lax.fori_loop(0, nrows * ncol_chunks, body, 0)


def _iota_fill(idx_ref, base):
    def body(j, _):
        idx_ref[pl.ds(j * 16, 16)] = lax.iota(jnp.int32, 16) + base + j * 16
        return 0

    lax.fori_loop(0, idx_ref.shape[0] // 16, body, 0)


def _fill2d(ref, nrows, val):
    ncol_chunks = ref.shape[1] // 16

    def body(i, _):
        r = i // ncol_chunks
        j = i % ncol_chunks
        ref[r, pl.ds(j * 16, 16)] = jnp.full((16,), val, jnp.float32)
        return 0

    lax.fori_loop(0, nrows * ncol_chunks, body, 0)


SCRUB_R = 12288


@functools.partial(
    pl.kernel,
    mesh=_mesh,
    out_type=jax.ShapeDtypeStruct((8,), jnp.float32),
    scratch_types=[
        pltpu.VMEM((K,), jnp.int32),
        pltpu.VMEM((K, D), jnp.float32),
        pltpu.VMEM_SHARED((SCRUB_R, D), jnp.float32),
    ],
)
def _scrub_kernel(out_hbm, iot_v, z_v, sp_sh):
    c = lax.axis_index("c")
    s = lax.axis_index("s")
    _fill2d(z_v, K, 0.0)
    base = s * (SCRUB_R // NS)
    for t in range(SCRUB_R // NS // K):
        _iota_fill(iot_v, base + t * K)
        pltpu.sync_copy(z_v, sp_sh.at[iot_v])

    @pl.when(jnp.logical_and(c == 0, s == 0))
    def _():
        pltpu.sync_copy(z_v.at[0, pl.ds(0, 8)], out_hbm)


@functools.partial(
    pl.kernel,
    mesh=_mesh,
    out_type=jax.ShapeDtypeStruct((NC * NP, D), jnp.float32),
    scratch_types=[
        pltpu.VMEM((K,), jnp.int32),
        pltpu.VMEM((K,), jnp.int32),
        pltpu.VMEM((K, D), jnp.float32),
        pltpu.VMEM((K, D), jnp.float32),
        pltpu.VMEM_SHARED((NP, D), jnp.float32),
        pltpu.SemaphoreType.DMA,
    ],
)
def _deg_kernel(dst_hbm, out_hbm, idx_v, iot_v, ones_v, stg_v, acc_sh, sem):
    c = lax.axis_index("c")
    s = lax.axis_index("s")
    w = s * NC + c

    _fill2d(ones_v, K, 1.0)
    for t in range(SEG // K):
        _iota_fill(iot_v, s * SEG + t * K)
        pltpu.async_copy(acc_sh.at[iot_v], stg_v, sem).wait()
        _neg2d(stg_v, K)
        pltpu.sync_copy(stg_v, acc_sh.at[iot_v], add=True)
    plsc.subcore_barrier()

    def body(i, _):
        g = i * NW + w
        pltpu.sync_copy(dst_hbm.at[pl.ds(g * K, K)], idx_v)
        pltpu.sync_copy(ones_v, acc_sh.at[idx_v], add=True)
        return 0

    lax.fori_loop(0, NB, body, 0)
    plsc.subcore_barrier()

    for t in range(SEG // K):
        _iota_fill(iot_v, s * SEG + t * K)
        pltpu.async_copy(acc_sh.at[iot_v], stg_v, sem).wait()
        pltpu.sync_copy(stg_v, out_hbm.at[pl.ds(c * NP + s * SEG + t * K, K)])


@functools.partial(
    pl.kernel,
    mesh=_mesh,
    out_type=jax.ShapeDtypeStruct((NC * NP, D), jnp.float32),
    scratch_types=[
        pltpu.VMEM((K,), jnp.int32),
        pltpu.VMEM((K,), jnp.int32),
        pltpu.VMEM((K,), jnp.int32),
        pltpu.VMEM((K,), jnp.int32),
        pltpu.VMEM((K,), jnp.int32),
        pltpu.VMEM((K, D), jnp.float32),
        pltpu.VMEM((K, D), jnp.float32),
        pltpu.VMEM_SHARED((NP, D), jnp.float32),
        pltpu.SemaphoreType.DMA,
        pltpu.SemaphoreType.DMA,
    ],
)
def _spmm_kernel(y_hbm, src_hbm, dst_hbm, out_hbm, didx_a, didx_b, sidx_a,
                 sidx_b, iot_v, rows_a, rows_b, acc_sh, sem_a, sem_b):
    c = lax.axis_index("c")
    s = lax.axis_index("s")
    w = s * NC + c

    for t in range(SEG // K):
        _iota_fill(iot_v, s * SEG + t * K)
        pltpu.async_copy(acc_sh.at[iot_v], rows_a, sem_a).wait()
        _neg2d(rows_a, K)
        pltpu.sync_copy(rows_a, acc_sh.at[iot_v], add=True)
    plsc.subcore_barrier()

    def stage(j, didx, sidx, rows, sem):
        g = j * NW + w
        pltpu.sync_copy(dst_hbm.at[pl.ds(g * K, K)], didx)
        pltpu.sync_copy(src_hbm.at[pl.ds(g * K, K)], sidx)
        return pltpu.async_copy(y_hbm.at[didx], rows, sem)

    def drain(cp, sidx, rows):
        cp.wait()
        pltpu.sync_copy(rows, acc_sh.at[sidx], add=True)

    stage(0, didx_a, sidx_a, rows_a, sem_a)

    def body(i, _):
        cb = stage(2 * i + 1, didx_b, sidx_b, rows_b, sem_b)
        pltpu.make_async_copy(y_hbm.at[pl.ds(0, K)], rows_a, sem_a).wait()
        pltpu.sync_copy(rows_a, acc_sh.at[sidx_a], add=True)
        stage(2 * i + 2, didx_a, sidx_a, rows_a, sem_a)
        cb.wait()
        pltpu.sync_copy(rows_b, acc_sh.at[sidx_b], add=True)
        return 0

    lax.fori_loop(0, (NB - 1) // 2, body, 0)
    drain(pltpu.make_async_copy(y_hbm.at[pl.ds(0, K)], rows_a, sem_a), sidx_a, rows_a)
    plsc.subcore_barrier()

    for t in range(SEG // K):
        _iota_fill(iot_v, s * SEG + t * K)
        pltpu.async_copy(acc_sh.at[iot_v], rows_a, sem_a).wait()
        pltpu.sync_copy(rows_a, out_hbm.at[pl.ds(c * NP + s * SEG + t * K, K)])


BM = 256
GRID = NP // BM


def _dinv_of(deg_ref):
    deg = deg_ref[0, :, 0:1] + deg_ref[1, :, 0:1] + 1.0
    return lax.rsqrt(deg)


def _tc1_body(deg_ref, x_ref, w1_ref, y_ref):
    dinv = _dinv_of(deg_ref)
    xw = jnp.dot(x_ref[...], w1_ref[...], preferred_element_type=jnp.float32,
                 precision=lax.Precision.HIGHEST)
    y_ref[...] = xw * dinv


def _tc2_body(deg_ref, y_ref, p_ref, b1_ref, g1_ref, bt1_ref, w2_ref, out_ref):
    dinv = _dinv_of(deg_ref)
    h = (y_ref[...] + p_ref[0] + p_ref[1]) * dinv + b1_ref[...]
    mu = jnp.mean(h, axis=1, keepdims=True)
    d = h - mu
    var = jnp.mean(d * d, axis=1, keepdims=True)
    hn = d * lax.rsqrt(var + 1e-5) * g1_ref[...] + bt1_ref[...]
    hr = jnp.maximum(hn, 0.0)
    out_ref[...] = jnp.dot(hr, w2_ref[...], preferred_element_type=jnp.float32,
                           precision=lax.Precision.HIGHEST) * dinv


def _tc3_body(deg_ref, y_ref, p_ref, b2_ref, out_ref):
    dinv = _dinv_of(deg_ref)
    out_ref[...] = (y_ref[...] + p_ref[0] + p_ref[1]) * dinv + b2_ref[...]


_deg_spec = pl.BlockSpec((NC, BM, D), lambda i: (0, i, 0))
_row_spec = pl.BlockSpec((BM, D), lambda i: (i, 0))
_p_spec = pl.BlockSpec((NC, BM, D), lambda i: (0, i, 0))
_w_spec = pl.BlockSpec((D, D), lambda i: (0, 0))
_v_spec = pl.BlockSpec((1, D), lambda i: (0, 0))
_out_sds = jax.ShapeDtypeStruct((NP, D), jnp.float32)

_DEBUG_JAX_SPMM = False


def kernel(x, edge_index, W1, b1, g1, beta1, W2, b2):
    src = jnp.pad(edge_index[0].astype(jnp.int32), (0, E_PAD - E), constant_values=N)
    dst = jnp.pad(edge_index[1].astype(jnp.int32), (0, E_PAD - E), constant_values=N)
    x_pad = jnp.pad(x, ((0, NP - N), (0, 0)))

    scrub = _scrub_kernel()
    dst = dst + (scrub[0] * 0.0).astype(jnp.int32)

    degp = _deg_kernel(dst).reshape(NC, NP, D)

    if _DEBUG_JAX_SPMM:
        def spmm(y):
            p = jnp.zeros((NP, D), jnp.float32).at[src].add(y[dst])
            return jnp.stack([p, jnp.zeros_like(p)])
    else:
        def spmm(y):
            return _spmm_kernel(y, src, dst).reshape(NC, NP, D)

    y1 = pl.pallas_call(
        _tc1_body,
        grid=(GRID,),
        in_specs=[_deg_spec, _row_spec, _w_spec],
        out_specs=_row_spec,
        out_shape=_out_sds,
    )(degp, x_pad, W1)

    p1 = spmm(y1)

    y2 = pl.pallas_call(
        _tc2_body,
        grid=(GRID,),
        in_specs=[_deg_spec, _row_spec, _p_spec, _v_spec, _v_spec, _v_spec, _w_spec],
        out_specs=_row_spec,
        out_shape=_out_sds,
    )(degp, y1, p1, b1.reshape(1, D), g1.reshape(1, D), beta1.reshape(1, D), W2)

    p2 = spmm(y2)

    out = pl.pallas_call(
        _tc3_body,
        grid=(GRID,),
        in_specs=[_deg_spec, _row_spec, _p_spec, _v_spec],
        out_specs=_row_spec,
        out_shape=_out_sds,
    )(degp, y2, p2, b2.reshape(1, D))

    return out[:N]
